# Initial kernel scaffold; baseline (speedup 1.0000x reference)
import jax
import jax.numpy as jnp
from jax.experimental import pallas as pl


def kernel(x, edge_index, edge_attr, batch, node_emb, We1, be1, W11, b11, W12, b12, We2, be2, W21, b21, W22, b22, Wlin, blin):
    raise NotImplementedError("write your pallas kernel here")


# speedup vs baseline: 2.6878x; 2.6878x over previous
import functools

import jax
import jax.numpy as jnp
from jax import lax
from jax.experimental import pallas as pl
from jax.experimental.pallas import tpu as pltpu
from jax.experimental.pallas import tpu_sc as plsc

_NG = 64

_F32 = jnp.float32


def _edge_body(ea_ref, ne_ref, we1_ref, be1_ref, we2_ref, be2_ref,
               m1_ref, e2_ref):
    a = ea_ref[...]
    m1 = jnp.dot(a, we1_ref[...], preferred_element_type=_F32)
    m1 = m1 + be1_ref[...] + ne_ref[...]
    m1_ref[...] = jnp.maximum(m1, 0.0)
    e2 = jnp.dot(a, we2_ref[...], preferred_element_type=_F32)
    e2_ref[...] = e2 + be2_ref[...]


def _edge_call(ea, ne, we1, be1, we2, be2, E, EDIM, HID, BE):
    nblk = E // BE
    grid = (2, nblk)
    return pl.pallas_call(
        _edge_body,
        grid=grid,
        in_specs=[
            pl.BlockSpec((BE, EDIM), lambda c, i: (i, 0)),
            pl.BlockSpec((1, HID // 2), lambda c, i: (0, c)),
            pl.BlockSpec((EDIM, HID // 2), lambda c, i: (0, c)),
            pl.BlockSpec((1, HID // 2), lambda c, i: (0, c)),
            pl.BlockSpec((EDIM, HID // 2), lambda c, i: (0, c)),
            pl.BlockSpec((1, HID // 2), lambda c, i: (0, c)),
        ],
        out_specs=[
            pl.BlockSpec((BE, HID // 2), lambda c, i, nblk=nblk: (c * nblk + i, 0)),
            pl.BlockSpec((BE, HID // 2), lambda c, i, nblk=nblk: (c * nblk + i, 0)),
        ],
        out_shape=[
            jax.ShapeDtypeStruct((2 * E, HID // 2), _F32),
            jax.ShapeDtypeStruct((2 * E, HID // 2), _F32),
        ],
    )(ea, ne, we1, be1, we2, be2)


def _sc_common_zero(acc, z_v, s, n, dc):
    nch = n // dc

    @pl.loop(0, dc)
    def _zero_rows(r):
        for j in range(8):
            z_v[r, pl.ds(j * 16, 16)] = jnp.zeros((16,), _F32)

    @pl.loop(0, (nch + 15) // 16)
    def _zero_acc(i):
        k = s + i * 16

        @pl.when(k < nch)
        def _():
            pltpu.sync_copy(z_v, acc.at[pl.ds(k * dc, dc)])


def _sc_common_drain(acc, z_v, out_hbm, c, s, n, dc):
    nch = n // dc

    @pl.loop(0, (nch + 15) // 16)
    def _drain(i):
        k = s + i * 16

        @pl.when(k < nch)
        def _():
            pltpu.sync_copy(acc.at[pl.ds(k * dc, dc)], z_v)
            pltpu.sync_copy(z_v, out_hbm.at[pl.ds(c * n + k * dc, dc)])


def _make_sc_scatter(N, E, EC, DC):
    EPT = E // 16
    NPT = N // 16
    NCH = EPT // EC
    mesh = plsc.VectorSubcoreMesh(core_axis_name="c", subcore_axis_name="s")

    @functools.partial(
        pl.kernel,
        out_type=jax.ShapeDtypeStruct((2 * N, 128), _F32),
        mesh=mesh,
        scratch_types=[
            pltpu.VMEM_SHARED((N, 128), _F32),
            pltpu.VMEM((EC,), jnp.int32),
            pltpu.VMEM((EC, 128), _F32),
            pltpu.VMEM((DC, 128), _F32),
        ],
    )
    def sc_scatter(msg_hbm, dst_hbm, out_hbm, acc, idx_v, row_v, z_v):
        c = lax.axis_index("c")
        s = lax.axis_index("s")
        _sc_common_zero(acc, z_v, s, N, DC)
        plsc.subcore_barrier()

        @pl.loop(0, NCH)
        def _chunk(ch):
            base = s * EPT + ch * EC
            pltpu.sync_copy(dst_hbm.at[pl.ds(base, EC)], idx_v)
            pltpu.sync_copy(msg_hbm.at[pl.ds(c * E + base, EC)], row_v)
            pltpu.sync_copy(row_v, acc.at[idx_v], add=True)

        plsc.subcore_barrier()
        _sc_common_drain(acc, z_v, out_hbm, c, s, N, DC)

    return sc_scatter


def _make_sc_gather_scatter(N, E, EC, DC):
    EPT = E // 16
    NPT = N // 16
    NCH = EPT // EC
    mesh = plsc.VectorSubcoreMesh(core_axis_name="c", subcore_axis_name="s")

    @functools.partial(
        pl.kernel,
        out_type=jax.ShapeDtypeStruct((2 * N, 128), _F32),
        mesh=mesh,
        scratch_types=[
            pltpu.VMEM_SHARED((N, 128), _F32),
            pltpu.VMEM((EC,), jnp.int32),
            pltpu.VMEM((EC,), jnp.int32),
            pltpu.VMEM((EC,), jnp.int32),
            pltpu.VMEM((EC, 128), _F32),
            pltpu.VMEM((EC, 128), _F32),
            pltpu.VMEM((DC, 128), _F32),
            pltpu.SemaphoreType.DMA,
        ],
    )
    def sc_gather_scatter(e2_hbm, h1v_hbm, src_hbm, dst_hbm, out_hbm,
                          acc, didx_v, sidx_v, gidx_v, e_v, h_v, z_v, sem):
        c = lax.axis_index("c")
        s = lax.axis_index("s")
        _sc_common_zero(acc, z_v, s, N, DC)
        plsc.subcore_barrier()

        @pl.loop(0, NCH)
        def _chunk(ch):
            base = s * EPT + ch * EC
            pltpu.sync_copy(src_hbm.at[pl.ds(base, EC)], sidx_v)
            pltpu.sync_copy(dst_hbm.at[pl.ds(base, EC)], didx_v)
            for j in range(EC // 16):
                sl = pl.ds(j * 16, 16)
                gidx_v[sl] = sidx_v[sl] * 2 + c
            gather = pltpu.async_copy(h1v_hbm.at[gidx_v], h_v, sem)
            pltpu.sync_copy(e2_hbm.at[pl.ds(c * E + base, EC)], e_v)
            gather.wait()

            @pl.loop(0, EC)
            def _rows(r):
                for j in range(8):
                    sl = pl.ds(j * 16, 16)
                    h_v[r, sl] = jnp.maximum(h_v[r, sl] + e_v[r, sl], 0.0)

            pltpu.sync_copy(h_v, acc.at[didx_v], add=True)

        plsc.subcore_barrier()
        _sc_common_drain(acc, z_v, out_hbm, c, s, N, DC)

    return sc_gather_scatter


def _mlp1_body(a0_ref, a1_ref, ne_ref, w1_ref, b1_ref, w2_ref, b2_ref, h_ref):
    h = ne_ref[...].shape[1] // 2
    u0 = a0_ref[...] + ne_ref[0:1, 0:h]
    u1 = a1_ref[...] + ne_ref[0:1, h:2 * h]
    t = (jnp.dot(u0, w1_ref[0:h, :], preferred_element_type=_F32)
         + jnp.dot(u1, w1_ref[h:2 * h, :], preferred_element_type=_F32)
         + b1_ref[...])
    t = jnp.maximum(t, 0.0)
    out = jnp.dot(t, w2_ref[...], preferred_element_type=_F32) + b2_ref[...]
    h_ref[...] = jnp.maximum(out, 0.0)


def _mlp1_call(agg1, ne, w1, b1, w2, b2, N, HID, BN):
    nblk = N // BN
    return pl.pallas_call(
        _mlp1_body,
        grid=(nblk,),
        in_specs=[
            pl.BlockSpec((BN, HID // 2), lambda i: (i, 0)),
            pl.BlockSpec((BN, HID // 2), lambda i, nblk=nblk: (nblk + i, 0)),
            pl.BlockSpec((1, HID), lambda i: (0, 0)),
            pl.BlockSpec((HID, HID), lambda i: (0, 0)),
            pl.BlockSpec((1, HID), lambda i: (0, 0)),
            pl.BlockSpec((HID, HID), lambda i: (0, 0)),
            pl.BlockSpec((1, HID), lambda i: (0, 0)),
        ],
        out_specs=pl.BlockSpec((BN, HID), lambda i: (i, 0)),
        out_shape=jax.ShapeDtypeStruct((N, HID), _F32),
    )(agg1, agg1, ne, w1, b1, w2, b2)


def _mlp2_body(h1_ref, a0_ref, a1_ref, bt_ref, w1_ref, b1_ref, w2_ref, b2_ref,
               wl_ref, bl_ref, h2_ref, out_ref, pool_ref, cnt_ref):
    i = pl.program_id(0)
    nprog = pl.num_programs(0)
    bn, hid = h1_ref[...].shape
    h = hid // 2

    u = h1_ref[...] + jnp.concatenate([a0_ref[...], a1_ref[...]], axis=1)
    t = jnp.dot(u, w1_ref[...], preferred_element_type=_F32) + b1_ref[...]
    t = jnp.maximum(t, 0.0)
    h2 = jnp.dot(t, w2_ref[...], preferred_element_type=_F32) + b2_ref[...]
    h2_ref[...] = h2

    groups = lax.broadcasted_iota(jnp.int32, (1, _NG), 1)
    onehot = (bt_ref[...] == groups).astype(_F32)
    pool_part = lax.dot_general(onehot, h2, (((0,), (0,)), ((), ())),
                                preferred_element_type=_F32)
    cnt_part = lax.dot_general(onehot, jnp.ones((bn, 128), _F32),
                               (((0,), (0,)), ((), ())),
                               preferred_element_type=_F32)

    @pl.when(i == 0)
    def _init():
        pool_ref[...] = jnp.zeros_like(pool_ref)
        cnt_ref[...] = jnp.zeros_like(cnt_ref)

    pool_ref[...] += pool_part
    cnt_ref[...] += cnt_part

    @pl.when(i == nprog - 1)
    def _final():
        outp = jnp.dot(pool_ref[...], wl_ref[...], preferred_element_type=_F32)
        cnt = jnp.maximum(cnt_ref[...], 1.0)
        out_ref[...] = outp / cnt + bl_ref[...]


def _mlp2_call(h1, agg2, batch2, w1, b1, w2, b2, wl, bl, N, HID, OUT, BN):
    nblk = N // BN
    return pl.pallas_call(
        _mlp2_body,
        grid=(nblk,),
        in_specs=[
            pl.BlockSpec((BN, HID), lambda i: (i, 0)),
            pl.BlockSpec((BN, HID // 2), lambda i: (i, 0)),
            pl.BlockSpec((BN, HID // 2), lambda i, nblk=nblk: (nblk + i, 0)),
            pl.BlockSpec((BN, 1), lambda i: (i, 0)),
            pl.BlockSpec((HID, HID), lambda i: (0, 0)),
            pl.BlockSpec((1, HID), lambda i: (0, 0)),
            pl.BlockSpec((HID, HID), lambda i: (0, 0)),
            pl.BlockSpec((1, HID), lambda i: (0, 0)),
            pl.BlockSpec((HID, OUT), lambda i: (0, 0)),
            pl.BlockSpec((1, OUT), lambda i: (0, 0)),
        ],
        out_specs=[
            pl.BlockSpec((BN, HID), lambda i: (i, 0)),
            pl.BlockSpec((_NG, OUT), lambda i: (0, 0)),
        ],
        out_shape=[
            jax.ShapeDtypeStruct((N, HID), _F32),
            jax.ShapeDtypeStruct((_NG, OUT), _F32),
        ],
        scratch_shapes=[
            pltpu.VMEM((_NG, HID), _F32),
            pltpu.VMEM((_NG, OUT), _F32),
        ],
    )(h1, agg2, agg2, batch2, w1, b1, w2, b2, wl, bl)


def kernel(x, edge_index, edge_attr, batch, node_emb, We1, be1, W11, b11,
           W12, b12, We2, be2, W21, b21, W22, b22, Wlin, blin):
    N = x.shape[0]
    E = edge_index.shape[1]
    EDIM = edge_attr.shape[1]
    HID = We1.shape[1]
    OUT = Wlin.shape[1]

    src = edge_index[0]
    dst = edge_index[1]
    ne = node_emb.reshape(1, HID)
    be1r = be1.reshape(1, HID)
    be2r = be2.reshape(1, HID)
    b11r = b11.reshape(1, HID)
    b12r = b12.reshape(1, HID)
    b21r = b21.reshape(1, HID)
    b22r = b22.reshape(1, HID)
    blinr = blin.reshape(1, OUT)

    BE = 2000
    BN = 2000
    EC = 80
    DC = 80

    msg1, e2 = _edge_call(edge_attr, ne, We1, be1r, We2, be2r, E, EDIM, HID, BE)

    sc1 = _make_sc_scatter(N, E, EC, DC)
    agg1 = sc1(msg1, dst)

    h1 = _mlp1_call(agg1, ne, W11, b11r, W12, b12r, N, HID, BN)
    h1v = h1.reshape(2 * N, HID // 2)

    sc2 = _make_sc_gather_scatter(N, E, EC, DC)
    agg2 = sc2(e2, h1v, src, dst)

    h2, out = _mlp2_call(h1, agg2, batch.reshape(N, 1), W21, b21r, W22, b22r,
                         Wlin, blinr, N, HID, OUT, BN)
    return (out, h2)


# speedup vs baseline: 3.8260x; 1.4235x over previous
import functools

import jax
import jax.numpy as jnp
from jax import lax
from jax.experimental import pallas as pl
from jax.experimental.pallas import tpu as pltpu
from jax.experimental.pallas import tpu_sc as plsc

_NG = 64

_F32 = jnp.float32


def _edge_msg1_body(ea_ref, ne_ref, we_ref, be_ref, o_ref):
    a = ea_ref[...]
    m1 = jnp.dot(a, we_ref[...], preferred_element_type=_F32)
    o_ref[...] = jnp.maximum(m1 + be_ref[...] + ne_ref[...], 0.0)


def _edge_e2_body(ea_ref, we_ref, be_ref, o_ref):
    a = ea_ref[...]
    o_ref[...] = jnp.dot(a, we_ref[...], preferred_element_type=_F32) + be_ref[...]


def _edge_call(body, args, E, EDIM, HID, BE, with_ne):
    nblk = E // BE
    half_spec = pl.BlockSpec((1, HID // 2), lambda c, i: (0, c))
    in_specs = [pl.BlockSpec((BE, EDIM), lambda c, i: (i, 0))]
    if with_ne:
        in_specs.append(half_spec)
    in_specs += [pl.BlockSpec((EDIM, HID // 2), lambda c, i: (0, c)), half_spec]
    return pl.pallas_call(
        body,
        grid=(2, nblk),
        in_specs=in_specs,
        out_specs=pl.BlockSpec((BE, HID // 2),
                               lambda c, i, nblk=nblk: (c * nblk + i, 0)),
        out_shape=jax.ShapeDtypeStruct((2 * E, HID // 2), _F32),
    )(*args)


def _sc_common_zero(acc, z_v, s, n, dc):
    nch = n // dc

    @pl.loop(0, dc)
    def _zero_rows(r):
        for j in range(8):
            z_v[r, pl.ds(j * 16, 16)] = jnp.zeros((16,), _F32)

    @pl.loop(0, (nch + 15) // 16)
    def _zero_acc(i):
        k = s + i * 16

        @pl.when(k < nch)
        def _():
            pltpu.sync_copy(z_v, acc.at[pl.ds(k * dc, dc)])


def _sc_common_drain(acc, z_v, out_hbm, c, s, n, dc):
    nch = n // dc

    @pl.loop(0, (nch + 15) // 16)
    def _drain(i):
        k = s + i * 16

        @pl.when(k < nch)
        def _():
            pltpu.sync_copy(acc.at[pl.ds(k * dc, dc)], z_v)
            pltpu.sync_copy(z_v, out_hbm.at[pl.ds(c * n + k * dc, dc)])


def _make_sc_scatter(N, E, EC, DC):
    EPT = E // 16
    NCH = EPT // EC
    TAIL = EPT - NCH * EC
    mesh = plsc.VectorSubcoreMesh(core_axis_name="c", subcore_axis_name="s")

    @functools.partial(
        pl.kernel,
        out_type=jax.ShapeDtypeStruct((2 * N, 128), _F32),
        mesh=mesh,
        scratch_types=[
            pltpu.VMEM_SHARED((N, 128), _F32),
            pltpu.VMEM((EC,), jnp.int32),
            pltpu.VMEM((EC,), jnp.int32),
            pltpu.VMEM((EC, 128), _F32),
            pltpu.VMEM((EC, 128), _F32),
            pltpu.VMEM((16,), jnp.int32),
            pltpu.VMEM((16, 128), _F32),
            pltpu.VMEM((DC, 128), _F32),
            pltpu.SemaphoreType.DMA,
            pltpu.SemaphoreType.DMA,
            pltpu.SemaphoreType.DMA,
            pltpu.SemaphoreType.DMA,
        ],
    )
    def sc_scatter(msg_hbm, dst_hbm, out_hbm, acc,
                   idx0, idx1, row0, row1, idx_t, row_t, z_v,
                   si0, si1, sr0, sr1):
        c = lax.axis_index("c")
        s = lax.axis_index("s")
        idx = (idx0, idx1)
        row = (row0, row1)
        sis = (si0, si1)
        srs = (sr0, sr1)
        _sc_common_zero(acc, z_v, s, N, DC)
        plsc.subcore_barrier()
        base0 = s * EPT

        def issue(k, p):
            pltpu.async_copy(dst_hbm.at[pl.ds(base0 + k * EC, EC)],
                             idx[p], sis[p])
            pltpu.async_copy(msg_hbm.at[pl.ds(c * E + base0 + k * EC, EC)],
                             row[p], srs[p])

        issue(0, 0)
        issue(1, 1)

        @pl.loop(0, NCH // 2)
        def _pair(i):
            for b in range(2):
                k = i * 2 + b
                pltpu.make_async_copy(dst_hbm.at[pl.ds(0, EC)],
                                      idx[b], sis[b]).wait()
                pltpu.make_async_copy(msg_hbm.at[pl.ds(0, EC)],
                                      row[b], srs[b]).wait()
                pltpu.sync_copy(row[b], acc.at[idx[b]], add=True)

                @pl.when(k + 2 < NCH)
                def _():
                    issue(k + 2, b)

        if TAIL:
            assert TAIL == 16
            tb = base0 + NCH * EC
            pltpu.sync_copy(dst_hbm.at[pl.ds(tb, TAIL)], idx_t)
            pltpu.sync_copy(msg_hbm.at[pl.ds(c * E + tb, TAIL)], row_t)
            pltpu.sync_copy(row_t, acc.at[idx_t], add=True)

        plsc.subcore_barrier()
        _sc_common_drain(acc, z_v, out_hbm, c, s, N, DC)

    return sc_scatter


def _make_sc_gather_scatter(N, E, EC, DC):
    EPT = E // 16
    NCH = EPT // EC
    TAIL = EPT - NCH * EC
    mesh = plsc.VectorSubcoreMesh(core_axis_name="c", subcore_axis_name="s")

    @functools.partial(
        pl.kernel,
        out_type=jax.ShapeDtypeStruct((2 * N, 128), _F32),
        mesh=mesh,
        scratch_types=[
            pltpu.VMEM_SHARED((N, 128), _F32),
            pltpu.VMEM((EC,), jnp.int32),
            pltpu.VMEM((EC,), jnp.int32),
            pltpu.VMEM((EC,), jnp.int32),
            pltpu.VMEM((EC,), jnp.int32),
            pltpu.VMEM((EC,), jnp.int32),
            pltpu.VMEM((EC,), jnp.int32),
            pltpu.VMEM((EC, 128), _F32),
            pltpu.VMEM((EC, 128), _F32),
            pltpu.VMEM((EC, 128), _F32),
            pltpu.VMEM((EC, 128), _F32),
            pltpu.VMEM((16,), jnp.int32),
            pltpu.VMEM((16,), jnp.int32),
            pltpu.VMEM((16, 128), _F32),
            pltpu.VMEM((16, 128), _F32),
            pltpu.VMEM((DC, 128), _F32),
            pltpu.SemaphoreType.DMA,
            pltpu.SemaphoreType.DMA,
            pltpu.SemaphoreType.DMA,
            pltpu.SemaphoreType.DMA,
            pltpu.SemaphoreType.DMA,
            pltpu.SemaphoreType.DMA,
            pltpu.SemaphoreType.DMA,
            pltpu.SemaphoreType.DMA,
        ],
    )
    def sc_gather_scatter(e2_hbm, h1v_hbm, src_hbm, dst_hbm, out_hbm,
                          acc, sidx0, sidx1, didx0, didx1, gidx0, gidx1,
                          e0, e1, h0, h1, sidx_t, didx_t, e_t, h_t, z_v,
                          ss0, ss1, sd0, sd1, se0, se1, sg0, sg1):
        c = lax.axis_index("c")
        s = lax.axis_index("s")
        sidx = (sidx0, sidx1)
        didx = (didx0, didx1)
        gidx = (gidx0, gidx1)
        ev = (e0, e1)
        hv = (h0, h1)
        sss = (ss0, ss1)
        sds = (sd0, sd1)
        ses = (se0, se1)
        sgs = (sg0, sg1)
        _sc_common_zero(acc, z_v, s, N, DC)
        plsc.subcore_barrier()
        base0 = s * EPT

        def issue_idx(k, p):
            pltpu.async_copy(src_hbm.at[pl.ds(base0 + k * EC, EC)],
                             sidx[p], sss[p])
            pltpu.async_copy(dst_hbm.at[pl.ds(base0 + k * EC, EC)],
                             didx[p], sds[p])
            pltpu.async_copy(e2_hbm.at[pl.ds(c * E + base0 + k * EC, EC)],
                             ev[p], ses[p])

        def start_gather(p):
            pltpu.make_async_copy(src_hbm.at[pl.ds(0, EC)],
                                  sidx[p], sss[p]).wait()
            for j in range(EC // 16):
                sl = pl.ds(j * 16, 16)
                gidx[p][sl] = sidx[p][sl] * 2 + c
            pltpu.async_copy(h1v_hbm.at[gidx[p]], hv[p], sgs[p])

        issue_idx(0, 0)
        issue_idx(1, 1)
        start_gather(0)

        @pl.loop(0, NCH // 2)
        def _pair(i):
            for b in range(2):
                k = i * 2 + b
                q = 1 - b

                @pl.when(k + 1 < NCH)
                def _():
                    start_gather(q)

                pltpu.make_async_copy(h1v_hbm.at[pl.ds(0, EC)],
                                      hv[b], sgs[b]).wait()
                pltpu.make_async_copy(e2_hbm.at[pl.ds(0, EC)],
                                      ev[b], ses[b]).wait()

                @pl.loop(0, EC)
                def _rows(r):
                    for j in range(8):
                        sl = pl.ds(j * 16, 16)
                        hv[b][r, sl] = jnp.maximum(
                            hv[b][r, sl] + ev[b][r, sl], 0.0)

                pltpu.make_async_copy(dst_hbm.at[pl.ds(0, EC)],
                                      didx[b], sds[b]).wait()
                pltpu.sync_copy(hv[b], acc.at[didx[b]], add=True)

                @pl.when(k + 2 < NCH)
                def _():
                    issue_idx(k + 2, b)

        if TAIL:
            assert TAIL == 16
            tb = base0 + NCH * EC
            pltpu.sync_copy(src_hbm.at[pl.ds(tb, TAIL)], sidx_t)
            sl = pl.ds(0, 16)
            didx_t[sl] = sidx_t[sl] * 2 + c
            pltpu.sync_copy(h1v_hbm.at[didx_t], h_t)
            pltpu.sync_copy(e2_hbm.at[pl.ds(c * E + tb, TAIL)], e_t)

            @pl.loop(0, TAIL)
            def _trows(r):
                for j in range(8):
                    sl2 = pl.ds(j * 16, 16)
                    h_t[r, sl2] = jnp.maximum(h_t[r, sl2] + e_t[r, sl2], 0.0)

            pltpu.sync_copy(dst_hbm.at[pl.ds(tb, TAIL)], didx_t)
            pltpu.sync_copy(h_t, acc.at[didx_t], add=True)

        plsc.subcore_barrier()
        _sc_common_drain(acc, z_v, out_hbm, c, s, N, DC)

    return sc_gather_scatter


def _mlp1_body(a0_ref, a1_ref, ne_ref, w1_ref, b1_ref, w2_ref, b2_ref, h_ref):
    h = ne_ref[...].shape[1] // 2
    u0 = a0_ref[...] + ne_ref[0:1, 0:h]
    u1 = a1_ref[...] + ne_ref[0:1, h:2 * h]
    t = (jnp.dot(u0, w1_ref[0:h, :], preferred_element_type=_F32)
         + jnp.dot(u1, w1_ref[h:2 * h, :], preferred_element_type=_F32)
         + b1_ref[...])
    t = jnp.maximum(t, 0.0)
    out = jnp.dot(t, w2_ref[...], preferred_element_type=_F32) + b2_ref[...]
    h_ref[...] = jnp.maximum(out, 0.0)


def _mlp1_call(agg1, ne, w1, b1, w2, b2, N, HID, BN):
    nblk = N // BN
    return pl.pallas_call(
        _mlp1_body,
        grid=(nblk,),
        in_specs=[
            pl.BlockSpec((BN, HID // 2), lambda i: (i, 0)),
            pl.BlockSpec((BN, HID // 2), lambda i, nblk=nblk: (nblk + i, 0)),
            pl.BlockSpec((1, HID), lambda i: (0, 0)),
            pl.BlockSpec((HID, HID), lambda i: (0, 0)),
            pl.BlockSpec((1, HID), lambda i: (0, 0)),
            pl.BlockSpec((HID, HID), lambda i: (0, 0)),
            pl.BlockSpec((1, HID), lambda i: (0, 0)),
        ],
        out_specs=pl.BlockSpec((BN, HID), lambda i: (i, 0)),
        out_shape=jax.ShapeDtypeStruct((N, HID), _F32),
    )(agg1, agg1, ne, w1, b1, w2, b2)


def _mlp2_body(h1_ref, a0_ref, a1_ref, bt_ref, w1_ref, b1_ref, w2_ref, b2_ref,
               wl_ref, bl_ref, h2_ref, out_ref, pool_ref, cnt_ref):
    i = pl.program_id(0)
    nprog = pl.num_programs(0)
    bn, hid = h1_ref[...].shape
    h = hid // 2

    u = h1_ref[...] + jnp.concatenate([a0_ref[...], a1_ref[...]], axis=1)
    t = jnp.dot(u, w1_ref[...], preferred_element_type=_F32) + b1_ref[...]
    t = jnp.maximum(t, 0.0)
    h2 = jnp.dot(t, w2_ref[...], preferred_element_type=_F32) + b2_ref[...]
    h2_ref[...] = h2

    groups = lax.broadcasted_iota(jnp.int32, (1, _NG), 1)
    onehot = (bt_ref[...] == groups).astype(_F32)
    pool_part = lax.dot_general(onehot, h2, (((0,), (0,)), ((), ())),
                                preferred_element_type=_F32)
    cnt_part = lax.dot_general(onehot, jnp.ones((bn, 128), _F32),
                               (((0,), (0,)), ((), ())),
                               preferred_element_type=_F32)

    @pl.when(i == 0)
    def _init():
        pool_ref[...] = jnp.zeros_like(pool_ref)
        cnt_ref[...] = jnp.zeros_like(cnt_ref)

    pool_ref[...] += pool_part
    cnt_ref[...] += cnt_part

    @pl.when(i == nprog - 1)
    def _final():
        outp = jnp.dot(pool_ref[...], wl_ref[...], preferred_element_type=_F32)
        cnt = jnp.maximum(cnt_ref[...], 1.0)
        out_ref[...] = outp / cnt + bl_ref[...]


def _mlp2_call(h1, agg2, batch2, w1, b1, w2, b2, wl, bl, N, HID, OUT, BN):
    nblk = N // BN
    return pl.pallas_call(
        _mlp2_body,
        grid=(nblk,),
        in_specs=[
            pl.BlockSpec((BN, HID), lambda i: (i, 0)),
            pl.BlockSpec((BN, HID // 2), lambda i: (i, 0)),
            pl.BlockSpec((BN, HID // 2), lambda i, nblk=nblk: (nblk + i, 0)),
            pl.BlockSpec((BN, 1), lambda i: (i, 0)),
            pl.BlockSpec((HID, HID), lambda i: (0, 0)),
            pl.BlockSpec((1, HID), lambda i: (0, 0)),
            pl.BlockSpec((HID, HID), lambda i: (0, 0)),
            pl.BlockSpec((1, HID), lambda i: (0, 0)),
            pl.BlockSpec((HID, OUT), lambda i: (0, 0)),
            pl.BlockSpec((1, OUT), lambda i: (0, 0)),
        ],
        out_specs=[
            pl.BlockSpec((BN, HID), lambda i: (i, 0)),
            pl.BlockSpec((_NG, OUT), lambda i: (0, 0)),
        ],
        out_shape=[
            jax.ShapeDtypeStruct((N, HID), _F32),
            jax.ShapeDtypeStruct((_NG, OUT), _F32),
        ],
        scratch_shapes=[
            pltpu.VMEM((_NG, HID), _F32),
            pltpu.VMEM((_NG, OUT), _F32),
        ],
    )(h1, agg2, agg2, batch2, w1, b1, w2, b2, wl, bl)


def kernel(x, edge_index, edge_attr, batch, node_emb, We1, be1, W11, b11,
           W12, b12, We2, be2, W21, b21, W22, b22, Wlin, blin):
    N = x.shape[0]
    E = edge_index.shape[1]
    EDIM = edge_attr.shape[1]
    HID = We1.shape[1]
    OUT = Wlin.shape[1]

    src = edge_index[0]
    dst = edge_index[1]
    ne = node_emb.reshape(1, HID)
    be1r = be1.reshape(1, HID)
    be2r = be2.reshape(1, HID)
    b11r = b11.reshape(1, HID)
    b12r = b12.reshape(1, HID)
    b21r = b21.reshape(1, HID)
    b22r = b22.reshape(1, HID)
    blinr = blin.reshape(1, OUT)

    BE = 2000
    BN = 2000
    EC = 128
    DC = 80

    msg1 = _edge_call(_edge_msg1_body, (edge_attr, ne, We1, be1r),
                      E, EDIM, HID, BE, with_ne=True)
    e2 = _edge_call(_edge_e2_body, (edge_attr, We2, be2r),
                    E, EDIM, HID, BE, with_ne=False)

    sc1 = _make_sc_scatter(N, E, EC, DC)
    agg1 = sc1(msg1, dst)
    del EC, DC

    h1 = _mlp1_call(agg1, ne, W11, b11r, W12, b12r, N, HID, BN)
    h1v = h1.reshape(2 * N, HID // 2)

    sc2 = _make_sc_gather_scatter(N, E, 64, 40)
    agg2 = sc2(e2, h1v, src, dst)

    h2, out = _mlp2_call(h1, agg2, batch.reshape(N, 1), W21, b21r, W22, b22r,
                         Wlin, blinr, N, HID, OUT, BN)
    return (out, h2)


# speedup vs baseline: 4.1274x; 1.0788x over previous
import functools

import jax
import jax.numpy as jnp
from jax import lax
from jax.experimental import pallas as pl
from jax.experimental.pallas import tpu as pltpu
from jax.experimental.pallas import tpu_sc as plsc

_NG = 64

_F32 = jnp.float32


def _edge_body(ea_ref, ne_ref, we1_ref, be1_ref, we2_ref, be2_ref,
               m1_ref, e2_ref):
    a = ea_ref[...]
    m1 = jnp.dot(a, we1_ref[...], preferred_element_type=_F32)
    m1_ref[...] = jnp.maximum(m1 + be1_ref[...] + ne_ref[...], 0.0)
    e2 = jnp.dot(a, we2_ref[...], preferred_element_type=_F32)
    e2_ref[...] = e2 + be2_ref[...]


def _edge_call(ea, ne, we1, be1, we2, be2, E, EDIM, HID, BE):
    nblk = E // BE
    half_spec = pl.BlockSpec((1, HID // 2), lambda c, i: (0, c))
    w_spec = pl.BlockSpec((EDIM, HID // 2), lambda c, i: (0, c))
    out_spec = pl.BlockSpec((BE, HID // 2),
                            lambda c, i, nblk=nblk: (c * nblk + i, 0))
    return pl.pallas_call(
        _edge_body,
        grid=(2, nblk),
        in_specs=[pl.BlockSpec((BE, EDIM), lambda c, i: (i, 0)),
                  half_spec, w_spec, half_spec, w_spec, half_spec],
        out_specs=[out_spec, out_spec],
        out_shape=[jax.ShapeDtypeStruct((2 * E, HID // 2), _F32),
                   jax.ShapeDtypeStruct((2 * E, HID // 2), _F32)],
    )(ea, ne, we1, be1, we2, be2)


def _sc_common_zero(acc, z_v, s, n, dc):
    nch = n // dc

    @pl.loop(0, dc)
    def _zero_rows(r):
        for j in range(8):
            z_v[r, pl.ds(j * 16, 16)] = jnp.zeros((16,), _F32)

    @pl.loop(0, (nch + 15) // 16)
    def _zero_acc(i):
        k = s + i * 16

        @pl.when(k < nch)
        def _():
            pltpu.sync_copy(z_v, acc.at[pl.ds(k * dc, dc)])


def _sc_common_drain(acc, z_v, out_hbm, c, s, n, dc):
    nch = n // dc

    @pl.loop(0, (nch + 15) // 16)
    def _drain(i):
        k = s + i * 16

        @pl.when(k < nch)
        def _():
            pltpu.sync_copy(acc.at[pl.ds(k * dc, dc)], z_v)
            pltpu.sync_copy(z_v, out_hbm.at[pl.ds(c * n + k * dc, dc)])


def _make_sc_scatter(N, E, EC, DC):
    EPT = E // 16
    NCH = EPT // EC
    TAIL = EPT - NCH * EC
    mesh = plsc.VectorSubcoreMesh(core_axis_name="c", subcore_axis_name="s")

    @functools.partial(
        pl.kernel,
        out_type=jax.ShapeDtypeStruct((2 * N, 128), _F32),
        mesh=mesh,
        scratch_types=[
            pltpu.VMEM_SHARED((N, 128), _F32),
            pltpu.VMEM((EC,), jnp.int32),
            pltpu.VMEM((EC,), jnp.int32),
            pltpu.VMEM((EC, 128), _F32),
            pltpu.VMEM((EC, 128), _F32),
            pltpu.VMEM((16,), jnp.int32),
            pltpu.VMEM((16, 128), _F32),
            pltpu.VMEM((DC, 128), _F32),
            pltpu.SemaphoreType.DMA,
            pltpu.SemaphoreType.DMA,
            pltpu.SemaphoreType.DMA,
            pltpu.SemaphoreType.DMA,
        ],
    )
    def sc_scatter(msg_hbm, dst_hbm, out_hbm, acc,
                   idx0, idx1, row0, row1, idx_t, row_t, z_v,
                   si0, si1, sr0, sr1):
        c = lax.axis_index("c")
        s = lax.axis_index("s")
        idx = (idx0, idx1)
        row = (row0, row1)
        sis = (si0, si1)
        srs = (sr0, sr1)
        _sc_common_zero(acc, z_v, s, N, DC)
        plsc.subcore_barrier()
        base0 = s * EPT

        def issue(k, p):
            pltpu.async_copy(dst_hbm.at[pl.ds(base0 + k * EC, EC)],
                             idx[p], sis[p])
            pltpu.async_copy(msg_hbm.at[pl.ds(c * E + base0 + k * EC, EC)],
                             row[p], srs[p])

        issue(0, 0)
        issue(1, 1)

        @pl.loop(0, NCH // 2)
        def _pair(i):
            for b in range(2):
                k = i * 2 + b
                pltpu.make_async_copy(dst_hbm.at[pl.ds(0, EC)],
                                      idx[b], sis[b]).wait()
                pltpu.make_async_copy(msg_hbm.at[pl.ds(0, EC)],
                                      row[b], srs[b]).wait()
                pltpu.sync_copy(row[b], acc.at[idx[b]], add=True)

                @pl.when(k + 2 < NCH)
                def _():
                    issue(k + 2, b)

        if TAIL:
            assert TAIL == 16
            tb = base0 + NCH * EC
            pltpu.sync_copy(dst_hbm.at[pl.ds(tb, TAIL)], idx_t)
            pltpu.sync_copy(msg_hbm.at[pl.ds(c * E + tb, TAIL)], row_t)
            pltpu.sync_copy(row_t, acc.at[idx_t], add=True)

        plsc.subcore_barrier()
        _sc_common_drain(acc, z_v, out_hbm, c, s, N, DC)

    return sc_scatter


def _make_sc_gather_scatter(N, E, EC, DC):
    EPT = E // 16
    NCH = EPT // EC
    TAIL = EPT - NCH * EC
    mesh = plsc.VectorSubcoreMesh(core_axis_name="c", subcore_axis_name="s")

    @functools.partial(
        pl.kernel,
        out_type=jax.ShapeDtypeStruct((2 * N, 128), _F32),
        mesh=mesh,
        scratch_types=[
            pltpu.VMEM_SHARED((N, 128), _F32),
            pltpu.VMEM((EC,), jnp.int32),
            pltpu.VMEM((EC,), jnp.int32),
            pltpu.VMEM((EC,), jnp.int32),
            pltpu.VMEM((EC,), jnp.int32),
            pltpu.VMEM((EC,), jnp.int32),
            pltpu.VMEM((EC,), jnp.int32),
            pltpu.VMEM((EC, 128), _F32),
            pltpu.VMEM((EC, 128), _F32),
            pltpu.VMEM((EC, 128), _F32),
            pltpu.VMEM((EC, 128), _F32),
            pltpu.VMEM((16,), jnp.int32),
            pltpu.VMEM((16,), jnp.int32),
            pltpu.VMEM((16, 128), _F32),
            pltpu.VMEM((16, 128), _F32),
            pltpu.VMEM((DC, 128), _F32),
            pltpu.SemaphoreType.DMA,
            pltpu.SemaphoreType.DMA,
            pltpu.SemaphoreType.DMA,
            pltpu.SemaphoreType.DMA,
            pltpu.SemaphoreType.DMA,
            pltpu.SemaphoreType.DMA,
            pltpu.SemaphoreType.DMA,
            pltpu.SemaphoreType.DMA,
        ],
    )
    def sc_gather_scatter(e2_hbm, h1v_hbm, src_hbm, dst_hbm, out_hbm,
                          acc, sidx0, sidx1, didx0, didx1, gidx0, gidx1,
                          e0, e1, h0, h1, sidx_t, didx_t, e_t, h_t, z_v,
                          ss0, ss1, sd0, sd1, se0, se1, sg0, sg1):
        c = lax.axis_index("c")
        s = lax.axis_index("s")
        sidx = (sidx0, sidx1)
        didx = (didx0, didx1)
        gidx = (gidx0, gidx1)
        ev = (e0, e1)
        hv = (h0, h1)
        sss = (ss0, ss1)
        sds = (sd0, sd1)
        ses = (se0, se1)
        sgs = (sg0, sg1)
        _sc_common_zero(acc, z_v, s, N, DC)
        plsc.subcore_barrier()
        base0 = s * EPT

        def issue_idx(k, p):
            pltpu.async_copy(src_hbm.at[pl.ds(base0 + k * EC, EC)],
                             sidx[p], sss[p])
            pltpu.async_copy(dst_hbm.at[pl.ds(base0 + k * EC, EC)],
                             didx[p], sds[p])
            pltpu.async_copy(e2_hbm.at[pl.ds(c * E + base0 + k * EC, EC)],
                             ev[p], ses[p])

        def start_gather(p):
            pltpu.make_async_copy(src_hbm.at[pl.ds(0, EC)],
                                  sidx[p], sss[p]).wait()
            for j in range(EC // 16):
                sl = pl.ds(j * 16, 16)
                gidx[p][sl] = sidx[p][sl] * 2 + c
            pltpu.async_copy(h1v_hbm.at[gidx[p]], hv[p], sgs[p])

        issue_idx(0, 0)
        issue_idx(1, 1)
        start_gather(0)

        @pl.loop(0, NCH // 2)
        def _pair(i):
            for b in range(2):
                k = i * 2 + b
                q = 1 - b

                @pl.when(k + 1 < NCH)
                def _():
                    start_gather(q)

                pltpu.make_async_copy(h1v_hbm.at[pl.ds(0, EC)],
                                      hv[b], sgs[b]).wait()
                pltpu.make_async_copy(e2_hbm.at[pl.ds(0, EC)],
                                      ev[b], ses[b]).wait()

                @pl.loop(0, EC)
                def _rows(r):
                    for j in range(8):
                        sl = pl.ds(j * 16, 16)
                        hv[b][r, sl] = jnp.maximum(
                            hv[b][r, sl] + ev[b][r, sl], 0.0)

                pltpu.make_async_copy(dst_hbm.at[pl.ds(0, EC)],
                                      didx[b], sds[b]).wait()
                pltpu.sync_copy(hv[b], acc.at[didx[b]], add=True)

                @pl.when(k + 2 < NCH)
                def _():
                    issue_idx(k + 2, b)

        if TAIL:
            assert TAIL == 16
            tb = base0 + NCH * EC
            pltpu.sync_copy(src_hbm.at[pl.ds(tb, TAIL)], sidx_t)
            sl = pl.ds(0, 16)
            didx_t[sl] = sidx_t[sl] * 2 + c
            pltpu.sync_copy(h1v_hbm.at[didx_t], h_t)
            pltpu.sync_copy(e2_hbm.at[pl.ds(c * E + tb, TAIL)], e_t)

            @pl.loop(0, TAIL)
            def _trows(r):
                for j in range(8):
                    sl2 = pl.ds(j * 16, 16)
                    h_t[r, sl2] = jnp.maximum(h_t[r, sl2] + e_t[r, sl2], 0.0)

            pltpu.sync_copy(dst_hbm.at[pl.ds(tb, TAIL)], didx_t)
            pltpu.sync_copy(h_t, acc.at[didx_t], add=True)

        plsc.subcore_barrier()
        _sc_common_drain(acc, z_v, out_hbm, c, s, N, DC)

    return sc_gather_scatter


def _mlp1_body(a0_ref, a1_ref, ne_ref, w1_ref, b1_ref, w2_ref, b2_ref, h_ref):
    h = ne_ref[...].shape[1] // 2
    u0 = a0_ref[...] + ne_ref[0:1, 0:h]
    u1 = a1_ref[...] + ne_ref[0:1, h:2 * h]
    t = (jnp.dot(u0, w1_ref[0:h, :], preferred_element_type=_F32)
         + jnp.dot(u1, w1_ref[h:2 * h, :], preferred_element_type=_F32)
         + b1_ref[...])
    t = jnp.maximum(t, 0.0)
    out = jnp.dot(t, w2_ref[...], preferred_element_type=_F32) + b2_ref[...]
    h_ref[...] = jnp.maximum(out, 0.0)


def _mlp1_call(agg1, ne, w1, b1, w2, b2, N, HID, BN):
    nblk = N // BN
    return pl.pallas_call(
        _mlp1_body,
        grid=(nblk,),
        in_specs=[
            pl.BlockSpec((BN, HID // 2), lambda i: (i, 0)),
            pl.BlockSpec((BN, HID // 2), lambda i, nblk=nblk: (nblk + i, 0)),
            pl.BlockSpec((1, HID), lambda i: (0, 0)),
            pl.BlockSpec((HID, HID), lambda i: (0, 0)),
            pl.BlockSpec((1, HID), lambda i: (0, 0)),
            pl.BlockSpec((HID, HID), lambda i: (0, 0)),
            pl.BlockSpec((1, HID), lambda i: (0, 0)),
        ],
        out_specs=pl.BlockSpec((BN, HID), lambda i: (i, 0)),
        out_shape=jax.ShapeDtypeStruct((N, HID), _F32),
    )(agg1, agg1, ne, w1, b1, w2, b2)


def _mlp2_body(h1_ref, a0_ref, a1_ref, bt_ref, w1_ref, b1_ref, w2_ref, b2_ref,
               wl_ref, bl_ref, h2_ref, out_ref, pool_ref, cnt_ref):
    i = pl.program_id(0)
    nprog = pl.num_programs(0)
    bn, hid = h1_ref[...].shape
    h = hid // 2

    u = h1_ref[...] + jnp.concatenate([a0_ref[...], a1_ref[...]], axis=1)
    t = jnp.dot(u, w1_ref[...], preferred_element_type=_F32) + b1_ref[...]
    t = jnp.maximum(t, 0.0)
    h2 = jnp.dot(t, w2_ref[...], preferred_element_type=_F32) + b2_ref[...]
    h2_ref[...] = h2

    groups = lax.broadcasted_iota(jnp.int32, (1, _NG), 1)
    onehot = (bt_ref[...] == groups).astype(_F32)
    pool_part = lax.dot_general(onehot, h2, (((0,), (0,)), ((), ())),
                                preferred_element_type=_F32)
    cnt_part = lax.dot_general(onehot, jnp.ones((bn, 128), _F32),
                               (((0,), (0,)), ((), ())),
                               preferred_element_type=_F32)

    @pl.when(i == 0)
    def _init():
        pool_ref[...] = jnp.zeros_like(pool_ref)
        cnt_ref[...] = jnp.zeros_like(cnt_ref)

    pool_ref[...] += pool_part
    cnt_ref[...] += cnt_part

    @pl.when(i == nprog - 1)
    def _final():
        outp = jnp.dot(pool_ref[...], wl_ref[...], preferred_element_type=_F32)
        cnt = jnp.maximum(cnt_ref[...], 1.0)
        out_ref[...] = outp / cnt + bl_ref[...]


def _mlp2_call(h1, agg2, batch2, w1, b1, w2, b2, wl, bl, N, HID, OUT, BN):
    nblk = N // BN
    return pl.pallas_call(
        _mlp2_body,
        grid=(nblk,),
        in_specs=[
            pl.BlockSpec((BN, HID), lambda i: (i, 0)),
            pl.BlockSpec((BN, HID // 2), lambda i: (i, 0)),
            pl.BlockSpec((BN, HID // 2), lambda i, nblk=nblk: (nblk + i, 0)),
            pl.BlockSpec((BN, 1), lambda i: (i, 0)),
            pl.BlockSpec((HID, HID), lambda i: (0, 0)),
            pl.BlockSpec((1, HID), lambda i: (0, 0)),
            pl.BlockSpec((HID, HID), lambda i: (0, 0)),
            pl.BlockSpec((1, HID), lambda i: (0, 0)),
            pl.BlockSpec((HID, OUT), lambda i: (0, 0)),
            pl.BlockSpec((1, OUT), lambda i: (0, 0)),
        ],
        out_specs=[
            pl.BlockSpec((BN, HID), lambda i: (i, 0)),
            pl.BlockSpec((_NG, OUT), lambda i: (0, 0)),
        ],
        out_shape=[
            jax.ShapeDtypeStruct((N, HID), _F32),
            jax.ShapeDtypeStruct((_NG, OUT), _F32),
        ],
        scratch_shapes=[
            pltpu.VMEM((_NG, HID), _F32),
            pltpu.VMEM((_NG, OUT), _F32),
        ],
    )(h1, agg2, agg2, batch2, w1, b1, w2, b2, wl, bl)


def kernel(x, edge_index, edge_attr, batch, node_emb, We1, be1, W11, b11,
           W12, b12, We2, be2, W21, b21, W22, b22, Wlin, blin):
    N = x.shape[0]
    E = edge_index.shape[1]
    EDIM = edge_attr.shape[1]
    HID = We1.shape[1]
    OUT = Wlin.shape[1]

    src = edge_index[0]
    dst = edge_index[1]
    ne = node_emb.reshape(1, HID)
    be1r = be1.reshape(1, HID)
    be2r = be2.reshape(1, HID)
    b11r = b11.reshape(1, HID)
    b12r = b12.reshape(1, HID)
    b21r = b21.reshape(1, HID)
    b22r = b22.reshape(1, HID)
    blinr = blin.reshape(1, OUT)

    BE = 2000
    BN = 2000
    EC = 128
    DC = 80

    msg1, e2 = _edge_call(edge_attr, ne, We1, be1r, We2, be2r,
                          E, EDIM, HID, BE)

    sc1 = _make_sc_scatter(N, E, EC, DC)
    agg1 = sc1(msg1, dst)
    del EC, DC

    h1 = _mlp1_call(agg1, ne, W11, b11r, W12, b12r, N, HID, BN)
    h1v = h1.reshape(2 * N, HID // 2)

    sc2 = _make_sc_gather_scatter(N, E, 64, 40)
    agg2 = sc2(e2, h1v, src, dst)

    h2, out = _mlp2_call(h1, agg2, batch.reshape(N, 1), W21, b21r, W22, b22r,
                         Wlin, blinr, N, HID, OUT, BN)
    return (out, h2)


# speedup vs baseline: 4.4474x; 1.0775x over previous
import functools

import numpy as np

import jax
import jax.numpy as jnp
from jax import lax
from jax.experimental import pallas as pl
from jax.experimental.pallas import tpu as pltpu
from jax.experimental.pallas import tpu_sc as plsc

_NG = 64

_F32 = jnp.float32


def _edge_body(ea_ref, ne_ref, we1_ref, be1_ref, we2_ref, be2_ref,
               m1_ref, e2_ref):
    a = ea_ref[...]
    m1 = jnp.dot(a, we1_ref[...], preferred_element_type=_F32)
    m1_ref[...] = jnp.maximum(m1 + be1_ref[...] + ne_ref[...], 0.0)
    e2 = jnp.dot(a, we2_ref[...], preferred_element_type=_F32)
    e2_ref[...] = e2 + be2_ref[...]


def _edge_call(ea, ne, we1, be1, we2, be2, E, EDIM, HID, BE):
    nblk = E // BE
    half_spec = pl.BlockSpec((1, HID // 2), lambda c, i: (0, c))
    w_spec = pl.BlockSpec((EDIM, HID // 2), lambda c, i: (0, c))
    out_spec = pl.BlockSpec((BE, HID // 2),
                            lambda c, i, nblk=nblk: (c * nblk + i, 0))
    return pl.pallas_call(
        _edge_body,
        grid=(2, nblk),
        in_specs=[pl.BlockSpec((BE, EDIM), lambda c, i: (i, 0)),
                  half_spec, w_spec, half_spec, w_spec, half_spec],
        out_specs=[out_spec, out_spec],
        out_shape=[jax.ShapeDtypeStruct((2 * E, HID // 2), _F32),
                   jax.ShapeDtypeStruct((2 * E, HID // 2), _F32)],
    )(ea, ne, we1, be1, we2, be2)


def _sc_common_zero(acc, zb, zrows, s, n, dc):
    nch = n // dc

    @pl.loop(0, zrows)
    def _zero_rows(r):
        for j in range(8):
            zb[r, pl.ds(j * 16, 16)] = jnp.zeros((16,), _F32)

    @pl.loop(0, (nch + 15) // 16)
    def _zero_acc(i):
        k = s + i * 16

        @pl.when(k < nch)
        def _():
            pltpu.sync_copy(zb.at[pl.ds(0, dc)], acc.at[pl.ds(k * dc, dc)])


def _sc_common_drain(acc, zb, out_hbm, c, s, n, dc):
    nch = n // dc

    @pl.loop(0, (nch + 15) // 16)
    def _drain(i):
        k = s + i * 16

        @pl.when(k < nch)
        def _():
            pltpu.sync_copy(acc.at[pl.ds(k * dc, dc)], zb.at[pl.ds(0, dc)])
            pltpu.sync_copy(zb.at[pl.ds(0, dc)],
                            out_hbm.at[pl.ds(c * n + k * dc, dc)])


def _make_sc_scatter(N, E, EC, DC):
    EPT = E // 16
    NCH = EPT // EC
    TAIL = EPT - NCH * EC
    mesh = plsc.VectorSubcoreMesh(core_axis_name="c", subcore_axis_name="s")

    @functools.partial(
        pl.kernel,
        out_type=jax.ShapeDtypeStruct((2 * N, 128), _F32),
        mesh=mesh,
        scratch_types=(
            [pltpu.VMEM_SHARED((N, 128), _F32)]
            + [pltpu.VMEM((EC,), jnp.int32)] * 3
            + [pltpu.VMEM((EC, 128), _F32)] * 3
            + [pltpu.VMEM((16,), jnp.int32),
               pltpu.VMEM((16, 128), _F32)]
            + [pltpu.SemaphoreType.DMA] * 9
        ),
    )
    def sc_scatter(msg_hbm, dst_hbm, out_hbm, acc,
                   idx0, idx1, idx2, row0, row1, row2, idx_t, row_t,
                   si0, si1, si2, sr0, sr1, sr2, sc0, sc1, sc2):
        c = lax.axis_index("c")
        s = lax.axis_index("s")
        idx = (idx0, idx1, idx2)
        row = (row0, row1, row2)
        sis = (si0, si1, si2)
        srs = (sr0, sr1, sr2)
        scs = (sc0, sc1, sc2)
        _sc_common_zero(acc, row0, EC, s, N, DC)
        plsc.subcore_barrier()
        base0 = s * EPT

        def issue(k, p):
            pltpu.async_copy(dst_hbm.at[pl.ds(base0 + k * EC, EC)],
                             idx[p], sis[p])
            pltpu.async_copy(msg_hbm.at[pl.ds(c * E + base0 + k * EC, EC)],
                             row[p], srs[p])

        def wait_scatter(p):
            pltpu.make_async_copy(row[p], acc.at[pl.ds(0, EC)], scs[p]).wait()

        issue(0, 0)
        issue(1, 1)

        @pl.loop(0, NCH // 3)
        def _trip(i):
            for b in range(3):
                k = i * 3 + b
                p2 = (b + 2) % 3
                pltpu.make_async_copy(dst_hbm.at[pl.ds(0, EC)],
                                      idx[b], sis[b]).wait()
                pltpu.make_async_copy(msg_hbm.at[pl.ds(0, EC)],
                                      row[b], srs[b]).wait()
                pltpu.async_copy(row[b], acc.at[idx[b]], scs[b], add=True)

                @pl.when(k >= 1)
                def _():
                    wait_scatter(p2)

                @pl.when(k + 2 < NCH)
                def _():
                    issue(k + 2, p2)

        wait_scatter((NCH - 1) % 3)

        if TAIL:
            assert TAIL == 16
            tb = base0 + NCH * EC
            pltpu.sync_copy(dst_hbm.at[pl.ds(tb, TAIL)], idx_t)
            pltpu.sync_copy(msg_hbm.at[pl.ds(c * E + tb, TAIL)], row_t)
            pltpu.sync_copy(row_t, acc.at[idx_t], add=True)

        plsc.subcore_barrier()
        _sc_common_drain(acc, row0, out_hbm, c, s, N, DC)

    return sc_scatter


def _make_sc_gather_scatter(N, E, EC, DC):
    EPT = E // 16
    NCH = EPT // EC
    TAIL = EPT - NCH * EC
    mesh = plsc.VectorSubcoreMesh(core_axis_name="c", subcore_axis_name="s")

    @functools.partial(
        pl.kernel,
        out_type=jax.ShapeDtypeStruct((2 * N, 128), _F32),
        mesh=mesh,
        scratch_types=(
            [pltpu.VMEM_SHARED((N, 128), _F32)]
            + [pltpu.VMEM((EC,), jnp.int32)] * 3
            + [pltpu.VMEM((EC,), jnp.int32)] * 3
            + [pltpu.VMEM((EC,), jnp.int32)] * 3
            + [pltpu.VMEM((EC, 128), _F32)] * 3
            + [pltpu.VMEM((EC, 128), _F32)] * 3
            + [pltpu.VMEM((16,), jnp.int32),
               pltpu.VMEM((16,), jnp.int32)]
            + [pltpu.SemaphoreType.DMA] * 15
        ),
    )
    def sc_gather_scatter(e2_hbm, h1v_hbm, src_hbm, dst_hbm, out_hbm,
                          acc, sidx0, sidx1, sidx2, didx0, didx1, didx2,
                          gidx0, gidx1, gidx2, e0, e1, e2b, h0, h1, h2b,
                          sidx_t, didx_t,
                          ss0, ss1, ss2, sd0, sd1, sd2,
                          se0, se1, se2, sg0, sg1, sg2, sa0, sa1, sa2):
        c = lax.axis_index("c")
        s = lax.axis_index("s")
        sidx = (sidx0, sidx1, sidx2)
        didx = (didx0, didx1, didx2)
        gidx = (gidx0, gidx1, gidx2)
        ev = (e0, e1, e2b)
        hv = (h0, h1, h2b)
        sss = (ss0, ss1, ss2)
        sds = (sd0, sd1, sd2)
        ses = (se0, se1, se2)
        sgs = (sg0, sg1, sg2)
        sscat = (sa0, sa1, sa2)
        _sc_common_zero(acc, h0, EC, s, N, DC)
        plsc.subcore_barrier()
        base0 = s * EPT

        def issue_idx(k, p):
            pltpu.async_copy(src_hbm.at[pl.ds(base0 + k * EC, EC)],
                             sidx[p], sss[p])
            pltpu.async_copy(dst_hbm.at[pl.ds(base0 + k * EC, EC)],
                             didx[p], sds[p])
            pltpu.async_copy(e2_hbm.at[pl.ds(c * E + base0 + k * EC, EC)],
                             ev[p], ses[p])

        def start_gather(p):
            pltpu.make_async_copy(src_hbm.at[pl.ds(0, EC)],
                                  sidx[p], sss[p]).wait()
            for j in range(EC // 16):
                sl = pl.ds(j * 16, 16)
                gidx[p][sl] = sidx[p][sl] * 2 + c
            pltpu.async_copy(h1v_hbm.at[gidx[p]], hv[p], sgs[p])

        def relu_add(h_ref, e_ref, nrows):
            @pl.loop(0, nrows)
            def _rows(r):
                for j in range(8):
                    sl = pl.ds(j * 16, 16)
                    h_ref[r, sl] = jnp.maximum(h_ref[r, sl] + e_ref[r, sl],
                                               0.0)

        issue_idx(0, 0)
        issue_idx(1, 1)
        start_gather(0)

        @pl.loop(0, NCH // 3)
        def _trip(i):
            for b in range(3):
                k = i * 3 + b
                p1 = (b + 1) % 3
                p2 = (b + 2) % 3

                @pl.when(k + 1 < NCH)
                def _():
                    start_gather(p1)

                pltpu.make_async_copy(h1v_hbm.at[pl.ds(0, EC)],
                                      hv[b], sgs[b]).wait()
                pltpu.make_async_copy(e2_hbm.at[pl.ds(0, EC)],
                                      ev[b], ses[b]).wait()
                relu_add(hv[b], ev[b], EC)
                pltpu.make_async_copy(dst_hbm.at[pl.ds(0, EC)],
                                      didx[b], sds[b]).wait()
                pltpu.async_copy(hv[b], acc.at[didx[b]], sscat[b], add=True)

                @pl.when(k >= 1)
                def _():
                    pltpu.make_async_copy(hv[p2], acc.at[pl.ds(0, EC)],
                                          sscat[p2]).wait()

                @pl.when(k + 2 < NCH)
                def _():
                    issue_idx(k + 2, p2)

        pltpu.make_async_copy(hv[(NCH - 1) % 3], acc.at[pl.ds(0, EC)],
                              sscat[(NCH - 1) % 3]).wait()

        if TAIL:
            assert TAIL == 16
            tb = base0 + NCH * EC
            pltpu.sync_copy(src_hbm.at[pl.ds(tb, TAIL)], sidx_t)
            sl = pl.ds(0, 16)
            didx_t[sl] = sidx_t[sl] * 2 + c
            pltpu.sync_copy(h1v_hbm.at[didx_t], h0.at[pl.ds(0, TAIL)])
            pltpu.sync_copy(e2_hbm.at[pl.ds(c * E + tb, TAIL)],
                            e0.at[pl.ds(0, TAIL)])
            relu_add(h0, e0, TAIL)
            pltpu.sync_copy(dst_hbm.at[pl.ds(tb, TAIL)], didx_t)
            pltpu.sync_copy(h0.at[pl.ds(0, TAIL)], acc.at[didx_t], add=True)

        plsc.subcore_barrier()
        _sc_common_drain(acc, h1, out_hbm, c, s, N, DC)

    return sc_gather_scatter


def _mlp1_body(a0_ref, a1_ref, ne_ref, w1_ref, b1_ref, w2_ref, b2_ref, h_ref):
    h = ne_ref[...].shape[1] // 2
    u0 = a0_ref[...] + ne_ref[0:1, 0:h]
    u1 = a1_ref[...] + ne_ref[0:1, h:2 * h]
    t = (jnp.dot(u0, w1_ref[0:h, :], preferred_element_type=_F32)
         + jnp.dot(u1, w1_ref[h:2 * h, :], preferred_element_type=_F32)
         + b1_ref[...])
    t = jnp.maximum(t, 0.0)
    out = jnp.dot(t, w2_ref[...], preferred_element_type=_F32) + b2_ref[...]
    h_ref[...] = jnp.maximum(out, 0.0)


def _mlp1_call(agg1, ne, w1, b1, w2, b2, N, HID, BN):
    nblk = N // BN
    return pl.pallas_call(
        _mlp1_body,
        grid=(nblk,),
        in_specs=[
            pl.BlockSpec((BN, HID // 2), lambda i: (i, 0)),
            pl.BlockSpec((BN, HID // 2), lambda i, nblk=nblk: (nblk + i, 0)),
            pl.BlockSpec((1, HID), lambda i: (0, 0)),
            pl.BlockSpec((HID, HID), lambda i: (0, 0)),
            pl.BlockSpec((1, HID), lambda i: (0, 0)),
            pl.BlockSpec((HID, HID), lambda i: (0, 0)),
            pl.BlockSpec((1, HID), lambda i: (0, 0)),
        ],
        out_specs=pl.BlockSpec((BN, HID), lambda i: (i, 0)),
        out_shape=jax.ShapeDtypeStruct((N, HID), _F32),
    )(agg1, agg1, ne, w1, b1, w2, b2)


def _mlp2_body(h1_ref, a0_ref, a1_ref, bt_ref, w1_ref, b1_ref, w2_ref, b2_ref,
               wl_ref, bl_ref, h2_ref, out_ref, pool_ref, cnt_ref):
    i = pl.program_id(0)
    nprog = pl.num_programs(0)
    bn, hid = h1_ref[...].shape
    h = hid // 2

    u = h1_ref[...] + jnp.concatenate([a0_ref[...], a1_ref[...]], axis=1)
    t = jnp.dot(u, w1_ref[...], preferred_element_type=_F32) + b1_ref[...]
    t = jnp.maximum(t, 0.0)
    h2 = jnp.dot(t, w2_ref[...], preferred_element_type=_F32) + b2_ref[...]
    h2_ref[...] = h2

    groups = lax.broadcasted_iota(jnp.int32, (1, _NG), 1)
    onehot = (bt_ref[...] == groups).astype(_F32)
    pool_part = lax.dot_general(onehot, h2, (((0,), (0,)), ((), ())),
                                preferred_element_type=_F32)
    cnt_part = lax.dot_general(onehot, jnp.ones((bn, 128), _F32),
                               (((0,), (0,)), ((), ())),
                               preferred_element_type=_F32)

    @pl.when(i == 0)
    def _init():
        pool_ref[...] = jnp.zeros_like(pool_ref)
        cnt_ref[...] = jnp.zeros_like(cnt_ref)

    pool_ref[...] += pool_part
    cnt_ref[...] += cnt_part

    @pl.when(i == nprog - 1)
    def _final():
        outp = jnp.dot(pool_ref[...], wl_ref[...], preferred_element_type=_F32)
        cnt = jnp.maximum(cnt_ref[...], 1.0)
        out_ref[...] = outp / cnt + bl_ref[...]


def _mlp2_call(h1, agg2, batch2, w1, b1, w2, b2, wl, bl, N, HID, OUT, BN):
    nblk = N // BN
    return pl.pallas_call(
        _mlp2_body,
        grid=(nblk,),
        in_specs=[
            pl.BlockSpec((BN, HID), lambda i: (i, 0)),
            pl.BlockSpec((BN, HID // 2), lambda i: (i, 0)),
            pl.BlockSpec((BN, HID // 2), lambda i, nblk=nblk: (nblk + i, 0)),
            pl.BlockSpec((BN, 1), lambda i: (i, 0)),
            pl.BlockSpec((HID, HID), lambda i: (0, 0)),
            pl.BlockSpec((1, HID), lambda i: (0, 0)),
            pl.BlockSpec((HID, HID), lambda i: (0, 0)),
            pl.BlockSpec((1, HID), lambda i: (0, 0)),
            pl.BlockSpec((HID, OUT), lambda i: (0, 0)),
            pl.BlockSpec((1, OUT), lambda i: (0, 0)),
        ],
        out_specs=[
            pl.BlockSpec((BN, HID), lambda i: (i, 0)),
            pl.BlockSpec((_NG, OUT), lambda i: (0, 0)),
        ],
        out_shape=[
            jax.ShapeDtypeStruct((N, HID), _F32),
            jax.ShapeDtypeStruct((_NG, OUT), _F32),
        ],
        scratch_shapes=[
            pltpu.VMEM((_NG, HID), _F32),
            pltpu.VMEM((_NG, OUT), _F32),
        ],
    )(h1, agg2, agg2, batch2, w1, b1, w2, b2, wl, bl)


def kernel(x, edge_index, edge_attr, batch, node_emb, We1, be1, W11, b11,
           W12, b12, We2, be2, W21, b21, W22, b22, Wlin, blin):
    N = x.shape[0]
    E = edge_index.shape[1]
    EDIM = edge_attr.shape[1]
    HID = We1.shape[1]
    OUT = Wlin.shape[1]

    src = edge_index[0]
    dst = edge_index[1]
    ne = node_emb.reshape(1, HID)
    be1r = be1.reshape(1, HID)
    be2r = be2.reshape(1, HID)
    b11r = b11.reshape(1, HID)
    b12r = b12.reshape(1, HID)
    b21r = b21.reshape(1, HID)
    b22r = b22.reshape(1, HID)
    blinr = blin.reshape(1, OUT)

    BE = 2000
    BN = 2000

    msg1, e2 = _edge_call(edge_attr, ne, We1, be1r, We2, be2r,
                          E, EDIM, HID, BE)

    sc1 = _make_sc_scatter(N, E, 64, 40)
    agg1 = sc1(msg1, dst)

    h1 = _mlp1_call(agg1, ne, W11, b11r, W12, b12r, N, HID, BN)
    h1v = h1.reshape(2 * N, HID // 2)

    sc2 = _make_sc_gather_scatter(N, E, 64, 40)
    agg2 = sc2(e2, h1v, src, dst)

    h2, out = _mlp2_call(h1, agg2, batch.reshape(N, 1), W21, b21r, W22, b22r,
                         Wlin, blinr, N, HID, OUT, BN)
    return (out, h2)


# speedup vs baseline: 4.4774x; 1.0067x over previous
import functools

import numpy as np

import jax
import jax.numpy as jnp
from jax import lax
from jax.experimental import pallas as pl
from jax.experimental.pallas import tpu as pltpu
from jax.experimental.pallas import tpu_sc as plsc

_NG = 64

_F32 = jnp.float32


def _edge_body(ea_ref, ne_ref, we1_ref, be1_ref, we2_ref, be2_ref,
               m1_ref, e2_ref):
    a = ea_ref[...]
    m1 = jnp.dot(a, we1_ref[...], preferred_element_type=_F32)
    m1_ref[...] = jnp.maximum(m1 + be1_ref[...] + ne_ref[...], 0.0)
    e2 = jnp.dot(a, we2_ref[...], preferred_element_type=_F32)
    e2_ref[...] = e2 + be2_ref[...]


def _edge_call(ea, ne, we1, be1, we2, be2, E, EDIM, HID, BE):
    nblk = E // BE
    half_spec = pl.BlockSpec((1, HID // 2), lambda c, i: (0, c))
    w_spec = pl.BlockSpec((EDIM, HID // 2), lambda c, i: (0, c))
    out_spec = pl.BlockSpec((BE, HID // 2),
                            lambda c, i, nblk=nblk: (c * nblk + i, 0))
    return pl.pallas_call(
        _edge_body,
        grid=(2, nblk),
        in_specs=[pl.BlockSpec((BE, EDIM), lambda c, i: (i, 0)),
                  half_spec, w_spec, half_spec, w_spec, half_spec],
        out_specs=[out_spec, out_spec],
        out_shape=[jax.ShapeDtypeStruct((2 * E, HID // 2), _F32),
                   jax.ShapeDtypeStruct((2 * E, HID // 2), _F32)],
    )(ea, ne, we1, be1, we2, be2)


def _sc_common_zero(acc, zb, zrows, s, n, dc):
    nch = n // dc

    @pl.loop(0, zrows)
    def _zero_rows(r):
        for j in range(8):
            zb[r, pl.ds(j * 16, 16)] = jnp.zeros((16,), _F32)

    @pl.loop(0, (nch + 15) // 16)
    def _zero_acc(i):
        k = s + i * 16

        @pl.when(k < nch)
        def _():
            pltpu.sync_copy(zb.at[pl.ds(0, dc)], acc.at[pl.ds(k * dc, dc)])


def _sc_common_drain(acc, out_hbm, c, s, n, dc, sem):
    nch = n // dc
    nit = (nch + 15) // 16

    @pl.loop(0, nit)
    def _drain(i):
        k = s + i * 16

        @pl.when(k < nch)
        def _():
            pltpu.async_copy(acc.at[pl.ds(k * dc, dc)],
                             out_hbm.at[pl.ds(c * n + k * dc, dc)], sem)

    @pl.loop(0, nit)
    def _drain_wait(i):
        k = s + i * 16

        @pl.when(k < nch)
        def _():
            pltpu.make_async_copy(acc.at[pl.ds(0, dc)],
                                  out_hbm.at[pl.ds(0, dc)], sem).wait()


def _make_sc_scatter(N, E, EC, DC):
    EPT = E // 16
    NCH = EPT // EC
    TAIL = EPT - NCH * EC
    mesh = plsc.VectorSubcoreMesh(core_axis_name="c", subcore_axis_name="s")

    @functools.partial(
        pl.kernel,
        out_type=jax.ShapeDtypeStruct((2 * N, 128), _F32),
        mesh=mesh,
        scratch_types=(
            [pltpu.VMEM_SHARED((N, 128), _F32)]
            + [pltpu.VMEM((EC,), jnp.int32)] * 3
            + [pltpu.VMEM((EC, 128), _F32)] * 3
            + [pltpu.VMEM((16,), jnp.int32),
               pltpu.VMEM((16, 128), _F32)]
            + [pltpu.SemaphoreType.DMA] * 10
        ),
    )
    def sc_scatter(msg_hbm, dst_hbm, out_hbm, acc,
                   idx0, idx1, idx2, row0, row1, row2, idx_t, row_t,
                   si0, si1, si2, sr0, sr1, sr2, sc0, sc1, sc2, sdr):
        c = lax.axis_index("c")
        s = lax.axis_index("s")
        idx = (idx0, idx1, idx2)
        row = (row0, row1, row2)
        sis = (si0, si1, si2)
        srs = (sr0, sr1, sr2)
        scs = (sc0, sc1, sc2)
        _sc_common_zero(acc, row0, EC, s, N, DC)
        plsc.subcore_barrier()
        base0 = s * EPT

        def issue(k, p):
            pltpu.async_copy(dst_hbm.at[pl.ds(base0 + k * EC, EC)],
                             idx[p], sis[p])
            pltpu.async_copy(msg_hbm.at[pl.ds(c * E + base0 + k * EC, EC)],
                             row[p], srs[p])

        def wait_scatter(p):
            pltpu.make_async_copy(row[p], acc.at[pl.ds(0, EC)], scs[p]).wait()

        issue(0, 0)
        issue(1, 1)

        @pl.loop(0, NCH // 3)
        def _trip(i):
            for b in range(3):
                k = i * 3 + b
                p2 = (b + 2) % 3
                pltpu.make_async_copy(dst_hbm.at[pl.ds(0, EC)],
                                      idx[b], sis[b]).wait()
                pltpu.make_async_copy(msg_hbm.at[pl.ds(0, EC)],
                                      row[b], srs[b]).wait()
                pltpu.async_copy(row[b], acc.at[idx[b]], scs[b], add=True)

                @pl.when(k >= 1)
                def _():
                    wait_scatter(p2)

                @pl.when(k + 2 < NCH)
                def _():
                    issue(k + 2, p2)

        wait_scatter((NCH - 1) % 3)

        if TAIL:
            assert TAIL == 16
            tb = base0 + NCH * EC
            pltpu.sync_copy(dst_hbm.at[pl.ds(tb, TAIL)], idx_t)
            pltpu.sync_copy(msg_hbm.at[pl.ds(c * E + tb, TAIL)], row_t)
            pltpu.sync_copy(row_t, acc.at[idx_t], add=True)

        plsc.subcore_barrier()
        _sc_common_drain(acc, out_hbm, c, s, N, DC, sdr)

    return sc_scatter


def _make_sc_gather_scatter(N, E, EC, DC):
    EPT = E // 16
    NCH = EPT // EC
    TAIL = EPT - NCH * EC
    mesh = plsc.VectorSubcoreMesh(core_axis_name="c", subcore_axis_name="s")

    @functools.partial(
        pl.kernel,
        out_type=jax.ShapeDtypeStruct((2 * N, 128), _F32),
        mesh=mesh,
        scratch_types=(
            [pltpu.VMEM_SHARED((N, 128), _F32)]
            + [pltpu.VMEM((EC,), jnp.int32)] * 3
            + [pltpu.VMEM((EC,), jnp.int32)] * 3
            + [pltpu.VMEM((EC,), jnp.int32)] * 3
            + [pltpu.VMEM((EC, 128), _F32)] * 3
            + [pltpu.VMEM((EC, 128), _F32)] * 3
            + [pltpu.VMEM((16,), jnp.int32),
               pltpu.VMEM((16,), jnp.int32)]
            + [pltpu.SemaphoreType.DMA] * 16
        ),
    )
    def sc_gather_scatter(e2_hbm, h1v_hbm, src_hbm, dst_hbm, out_hbm,
                          acc, sidx0, sidx1, sidx2, didx0, didx1, didx2,
                          gidx0, gidx1, gidx2, e0, e1, e2b, h0, h1, h2b,
                          sidx_t, didx_t,
                          ss0, ss1, ss2, sd0, sd1, sd2,
                          se0, se1, se2, sg0, sg1, sg2, sa0, sa1, sa2, sdr):
        c = lax.axis_index("c")
        s = lax.axis_index("s")
        sidx = (sidx0, sidx1, sidx2)
        didx = (didx0, didx1, didx2)
        gidx = (gidx0, gidx1, gidx2)
        ev = (e0, e1, e2b)
        hv = (h0, h1, h2b)
        sss = (ss0, ss1, ss2)
        sds = (sd0, sd1, sd2)
        ses = (se0, se1, se2)
        sgs = (sg0, sg1, sg2)
        sscat = (sa0, sa1, sa2)
        _sc_common_zero(acc, h0, EC, s, N, DC)
        plsc.subcore_barrier()
        base0 = s * EPT

        def issue_idx(k, p):
            pltpu.async_copy(src_hbm.at[pl.ds(base0 + k * EC, EC)],
                             sidx[p], sss[p])
            pltpu.async_copy(dst_hbm.at[pl.ds(base0 + k * EC, EC)],
                             didx[p], sds[p])
            pltpu.async_copy(e2_hbm.at[pl.ds(c * E + base0 + k * EC, EC)],
                             ev[p], ses[p])

        def start_gather(p):
            pltpu.make_async_copy(src_hbm.at[pl.ds(0, EC)],
                                  sidx[p], sss[p]).wait()
            for j in range(EC // 16):
                sl = pl.ds(j * 16, 16)
                gidx[p][sl] = sidx[p][sl] * 2 + c
            pltpu.async_copy(h1v_hbm.at[gidx[p]], hv[p], sgs[p])

        def relu_add(h_ref, e_ref, nrows):
            @pl.loop(0, nrows)
            def _rows(r):
                for j in range(8):
                    sl = pl.ds(j * 16, 16)
                    h_ref[r, sl] = jnp.maximum(h_ref[r, sl] + e_ref[r, sl],
                                               0.0)

        issue_idx(0, 0)
        issue_idx(1, 1)
        start_gather(0)

        @pl.loop(0, NCH // 3)
        def _trip(i):
            for b in range(3):
                k = i * 3 + b
                p1 = (b + 1) % 3
                p2 = (b + 2) % 3

                @pl.when(k + 1 < NCH)
                def _():
                    start_gather(p1)

                pltpu.make_async_copy(h1v_hbm.at[pl.ds(0, EC)],
                                      hv[b], sgs[b]).wait()
                pltpu.make_async_copy(e2_hbm.at[pl.ds(0, EC)],
                                      ev[b], ses[b]).wait()
                relu_add(hv[b], ev[b], EC)
                pltpu.make_async_copy(dst_hbm.at[pl.ds(0, EC)],
                                      didx[b], sds[b]).wait()
                pltpu.async_copy(hv[b], acc.at[didx[b]], sscat[b], add=True)

                @pl.when(k >= 1)
                def _():
                    pltpu.make_async_copy(hv[p2], acc.at[pl.ds(0, EC)],
                                          sscat[p2]).wait()

                @pl.when(k + 2 < NCH)
                def _():
                    issue_idx(k + 2, p2)

        pltpu.make_async_copy(hv[(NCH - 1) % 3], acc.at[pl.ds(0, EC)],
                              sscat[(NCH - 1) % 3]).wait()

        if TAIL:
            assert TAIL == 16
            tb = base0 + NCH * EC
            pltpu.sync_copy(src_hbm.at[pl.ds(tb, TAIL)], sidx_t)
            sl = pl.ds(0, 16)
            didx_t[sl] = sidx_t[sl] * 2 + c
            pltpu.sync_copy(h1v_hbm.at[didx_t], h0.at[pl.ds(0, TAIL)])
            pltpu.sync_copy(e2_hbm.at[pl.ds(c * E + tb, TAIL)],
                            e0.at[pl.ds(0, TAIL)])
            relu_add(h0, e0, TAIL)
            pltpu.sync_copy(dst_hbm.at[pl.ds(tb, TAIL)], didx_t)
            pltpu.sync_copy(h0.at[pl.ds(0, TAIL)], acc.at[didx_t], add=True)

        plsc.subcore_barrier()
        _sc_common_drain(acc, out_hbm, c, s, N, DC, sdr)

    return sc_gather_scatter


def _mlp1_body(a0_ref, a1_ref, ne_ref, w1_ref, b1_ref, w2_ref, b2_ref, h_ref):
    h = ne_ref[...].shape[1] // 2
    u0 = a0_ref[...] + ne_ref[0:1, 0:h]
    u1 = a1_ref[...] + ne_ref[0:1, h:2 * h]
    t = (jnp.dot(u0, w1_ref[0:h, :], preferred_element_type=_F32)
         + jnp.dot(u1, w1_ref[h:2 * h, :], preferred_element_type=_F32)
         + b1_ref[...])
    t = jnp.maximum(t, 0.0)
    out = jnp.dot(t, w2_ref[...], preferred_element_type=_F32) + b2_ref[...]
    h_ref[...] = jnp.maximum(out, 0.0)


def _mlp1_call(agg1, ne, w1, b1, w2, b2, N, HID, BN):
    nblk = N // BN
    return pl.pallas_call(
        _mlp1_body,
        grid=(nblk,),
        in_specs=[
            pl.BlockSpec((BN, HID // 2), lambda i: (i, 0)),
            pl.BlockSpec((BN, HID // 2), lambda i, nblk=nblk: (nblk + i, 0)),
            pl.BlockSpec((1, HID), lambda i: (0, 0)),
            pl.BlockSpec((HID, HID), lambda i: (0, 0)),
            pl.BlockSpec((1, HID), lambda i: (0, 0)),
            pl.BlockSpec((HID, HID), lambda i: (0, 0)),
            pl.BlockSpec((1, HID), lambda i: (0, 0)),
        ],
        out_specs=pl.BlockSpec((BN, HID), lambda i: (i, 0)),
        out_shape=jax.ShapeDtypeStruct((N, HID), _F32),
    )(agg1, agg1, ne, w1, b1, w2, b2)


def _mlp2_body(h1_ref, a0_ref, a1_ref, bt_ref, w1_ref, b1_ref, w2_ref, b2_ref,
               wl_ref, bl_ref, h2_ref, out_ref, pool_ref, cnt_ref):
    i = pl.program_id(0)
    nprog = pl.num_programs(0)
    bn, hid = h1_ref[...].shape
    h = hid // 2

    u = h1_ref[...] + jnp.concatenate([a0_ref[...], a1_ref[...]], axis=1)
    t = jnp.dot(u, w1_ref[...], preferred_element_type=_F32) + b1_ref[...]
    t = jnp.maximum(t, 0.0)
    h2 = jnp.dot(t, w2_ref[...], preferred_element_type=_F32) + b2_ref[...]
    h2_ref[...] = h2

    groups = lax.broadcasted_iota(jnp.int32, (1, _NG), 1)
    onehot = (bt_ref[...] == groups).astype(_F32)
    pool_part = lax.dot_general(onehot, h2, (((0,), (0,)), ((), ())),
                                preferred_element_type=_F32)
    cnt_part = lax.dot_general(onehot, jnp.ones((bn, 128), _F32),
                               (((0,), (0,)), ((), ())),
                               preferred_element_type=_F32)

    @pl.when(i == 0)
    def _init():
        pool_ref[...] = jnp.zeros_like(pool_ref)
        cnt_ref[...] = jnp.zeros_like(cnt_ref)

    pool_ref[...] += pool_part
    cnt_ref[...] += cnt_part

    @pl.when(i == nprog - 1)
    def _final():
        outp = jnp.dot(pool_ref[...], wl_ref[...], preferred_element_type=_F32)
        cnt = jnp.maximum(cnt_ref[...], 1.0)
        out_ref[...] = outp / cnt + bl_ref[...]


def _mlp2_call(h1, agg2, batch2, w1, b1, w2, b2, wl, bl, N, HID, OUT, BN):
    nblk = N // BN
    return pl.pallas_call(
        _mlp2_body,
        grid=(nblk,),
        in_specs=[
            pl.BlockSpec((BN, HID), lambda i: (i, 0)),
            pl.BlockSpec((BN, HID // 2), lambda i: (i, 0)),
            pl.BlockSpec((BN, HID // 2), lambda i, nblk=nblk: (nblk + i, 0)),
            pl.BlockSpec((BN, 1), lambda i: (i, 0)),
            pl.BlockSpec((HID, HID), lambda i: (0, 0)),
            pl.BlockSpec((1, HID), lambda i: (0, 0)),
            pl.BlockSpec((HID, HID), lambda i: (0, 0)),
            pl.BlockSpec((1, HID), lambda i: (0, 0)),
            pl.BlockSpec((HID, OUT), lambda i: (0, 0)),
            pl.BlockSpec((1, OUT), lambda i: (0, 0)),
        ],
        out_specs=[
            pl.BlockSpec((BN, HID), lambda i: (i, 0)),
            pl.BlockSpec((_NG, OUT), lambda i: (0, 0)),
        ],
        out_shape=[
            jax.ShapeDtypeStruct((N, HID), _F32),
            jax.ShapeDtypeStruct((_NG, OUT), _F32),
        ],
        scratch_shapes=[
            pltpu.VMEM((_NG, HID), _F32),
            pltpu.VMEM((_NG, OUT), _F32),
        ],
    )(h1, agg2, agg2, batch2, w1, b1, w2, b2, wl, bl)


def kernel(x, edge_index, edge_attr, batch, node_emb, We1, be1, W11, b11,
           W12, b12, We2, be2, W21, b21, W22, b22, Wlin, blin):
    N = x.shape[0]
    E = edge_index.shape[1]
    EDIM = edge_attr.shape[1]
    HID = We1.shape[1]
    OUT = Wlin.shape[1]

    src = edge_index[0]
    dst = edge_index[1]
    ne = node_emb.reshape(1, HID)
    be1r = be1.reshape(1, HID)
    be2r = be2.reshape(1, HID)
    b11r = b11.reshape(1, HID)
    b12r = b12.reshape(1, HID)
    b21r = b21.reshape(1, HID)
    b22r = b22.reshape(1, HID)
    blinr = blin.reshape(1, OUT)

    BE = 2000
    BN = 2000

    msg1, e2 = _edge_call(edge_attr, ne, We1, be1r, We2, be2r,
                          E, EDIM, HID, BE)

    sc1 = _make_sc_scatter(N, E, 64, 40)
    agg1 = sc1(msg1, dst)

    h1 = _mlp1_call(agg1, ne, W11, b11r, W12, b12r, N, HID, BN)
    h1v = h1.reshape(2 * N, HID // 2)

    sc2 = _make_sc_gather_scatter(N, E, 64, 40)
    agg2 = sc2(e2, h1v, src, dst)

    h2, out = _mlp2_call(h1, agg2, batch.reshape(N, 1), W21, b21r, W22, b22r,
                         Wlin, blinr, N, HID, OUT, BN)
    return (out, h2)
